# Initial kernel scaffold; baseline (speedup 1.0000x reference)
import jax
import jax.numpy as jnp
from jax.experimental import pallas as pl


def kernel(nfeat, edge_index, erel, emonth, cat_edge_index, cat_embedding, rel_emb, norm_emb, month_emb, W_tn, b_tn, W_tr, b_tr, W_te, b_te):
    raise NotImplementedError("write your pallas kernel here")


# speedup vs baseline: 7.3958x; 7.3958x over previous
import functools
import jax
import jax.numpy as jnp
from jax import lax
from jax.experimental import pallas as pl
from jax.experimental.pallas import tpu as pltpu
from jax.experimental.pallas import tpu_sc as plsc

N = 10000
E = 320000
D = 128
NC = 500
EC = 100000
ECP = 102400
SW = 48
NW = 32
EB = 80
CB = 80
EPW = E // NW
CPW = ECP // NW
NP1 = 10240

_mesh = plsc.VectorSubcoreMesh(core_axis_name="c", subcore_axis_name="s")


def _stable_body(x_ref, p_ref, o_ref):
    x = x_ref[...]
    s = jnp.dot(x, p_ref[...], preferred_element_type=jnp.float32)
    nn = jnp.sum(x * x, axis=1, keepdims=True)
    col = lax.broadcasted_iota(jnp.int32, s.shape, 1)
    o_ref[...] = jnp.where(col == 38, nn, s)


def _node_table(nfeat, P):
    nblk = 1000
    return pl.pallas_call(
        _stable_body,
        grid=(N // nblk,),
        in_specs=[
            pl.BlockSpec((nblk, D), lambda i: (i, 0)),
            pl.BlockSpec((D, SW), lambda i: (0, 0)),
        ],
        out_specs=pl.BlockSpec((nblk, SW), lambda i: (i, 0)),
        out_shape=jax.ShapeDtypeStruct((N, SW), jnp.float32),
    )(nfeat, P)


def _edge_body(nfeat_hbm, sflat_hbm, src_hbm, dst_hbm, rel_hbm, mon_hbm,
               crel_hbm, cmon_hbm, z2_hbm, z1_hbm,
               numer_hbm, denom_hbm,
               numer_sh, denom_sh,
               h_v, t_v, srcv, dstv, relv, monv, finv,
               ia0, ia1, ia2, ia3, ia4, ia5, ia6, ia7, ia8, ia9,
               va0, va1, va2, va3, va4, va5, va6, va7, va8, va9,
               crel_v, cmon_v, sem):
    cid = lax.axis_index("c")
    sid = lax.axis_index("s")

    @pl.when(sid == 0)
    def _init():
        pltpu.sync_copy(z2_hbm.at[pl.ds(0, N)], numer_sh)
        pltpu.sync_copy(z1_hbm, denom_sh)

    plsc.subcore_barrier()

    pltpu.sync_copy(crel_hbm, crel_v)
    pltpu.sync_copy(cmon_hbm, cmon_v)
    c1vec = crel_v[0, :]
    c2vec = crel_v[1, :]
    c3vec = crel_v[2, :]
    d1vec = cmon_v[0, :]
    d2vec = cmon_v[1, :]
    d3vec = cmon_v[2, :]
    lane = lax.iota(jnp.int32, 16)
    p8 = lane ^ 8
    p4 = lane ^ 4
    p2 = lane ^ 2
    p1 = lane ^ 1
    zlane = lane * 0

    wid = sid * 2 + cid
    base0 = wid * EPW

    def chunk(i, carry):
        base = base0 + i * EB
        pltpu.sync_copy(src_hbm.at[pl.ds(base, EB)], srcv)
        pltpu.sync_copy(dst_hbm.at[pl.ds(base, EB)], dstv)
        pltpu.sync_copy(rel_hbm.at[pl.ds(base, EB)], relv)
        pltpu.sync_copy(mon_hbm.at[pl.ds(base, EB)], monv)

        def bidx(g, c):
            sl = pl.ds(g * 16, 16)
            s16 = srcv[sl] * SW
            d16 = dstv[sl] * SW
            r16 = relv[sl]
            m16 = monv[sl]
            ia0[sl] = s16 + r16
            ia1[sl] = d16 + r16
            ia2[sl] = s16 + (r16 + 7)
            ia3[sl] = d16 + (r16 + 7)
            ia4[sl] = s16 + (m16 + 14)
            ia5[sl] = d16 + (m16 + 14)
            ia6[sl] = s16 + (m16 + 26)
            ia7[sl] = d16 + (m16 + 26)
            ia8[sl] = s16 + 38
            ia9[sl] = d16 + 38
            return c

        lax.fori_loop(0, EB // 16, bidx, 0)

        cps = [pltpu.async_copy(nfeat_hbm.at[srcv], h_v, sem),
               pltpu.async_copy(nfeat_hbm.at[dstv], t_v, sem)]
        for iref, vref in ((ia0, va0), (ia1, va1), (ia2, va2), (ia3, va3),
                           (ia4, va4), (ia5, va5), (ia6, va6), (ia7, va7),
                           (ia8, va8), (ia9, va9)):
            cps.append(pltpu.async_copy(sflat_hbm.at[iref], vref, sem))
        for cp in cps:
            cp.wait()

        def group(g, c):
            e0 = g * 16
            sl = pl.ds(e0, 16)
            dot16 = jnp.zeros((16,), jnp.float32)
            for l in range(16):
                acc = jnp.zeros((16,), jnp.float32)
                for j in range(8):
                    dsl = pl.ds(16 * j, 16)
                    acc = acc + h_v[e0 + l, dsl] * t_v[e0 + l, dsl]
                acc = acc + acc.at[p8].get(mode="promise_in_bounds")
                acc = acc + acc.at[p4].get(mode="promise_in_bounds")
                acc = acc + acc.at[p2].get(mode="promise_in_bounds")
                acc = acc + acc.at[p1].get(mode="promise_in_bounds")
                dot16 = jnp.where(lane == l, acc, dot16)
            r16 = relv[sl]
            m16 = monv[sl]
            c1v = c1vec.at[r16].get(mode="promise_in_bounds")
            c2v = c2vec.at[r16].get(mode="promise_in_bounds")
            c3v = c3vec.at[r16].get(mode="promise_in_bounds")
            d1v = d1vec.at[m16].get(mode="promise_in_bounds")
            d2v = d2vec.at[m16].get(mode="promise_in_bounds")
            d3v = d3vec.at[m16].get(mode="promise_in_bounds")
            q = va8[sl] + va9[sl] - 2.0 * dot16
            su = va0[sl] - va1[sl]
            url = va2[sl] - va3[sl]
            stn = va4[sl] - va5[sl]
            utr = va6[sl] - va7[sl]
            dist1 = q + su * su * c1v + c2v + 2.0 * url - 2.0 * su * c3v
            dist2 = q + stn * stn * d1v + d2v + 2.0 * utr - 2.0 * stn * d3v
            fin16 = jnp.exp(-(dist1 + dist2))
            finv[sl] = fin16
            for l in range(16):
                fb = fin16.at[zlane + l].get(mode="promise_in_bounds")
                for j in range(8):
                    dsl = pl.ds(16 * j, 16)
                    h_v[e0 + l, dsl] = h_v[e0 + l, dsl] * fb
            return c

        lax.fori_loop(0, EB // 16, group, 0)
        pltpu.sync_copy(h_v, numer_sh.at[dstv], add=True)
        pltpu.sync_copy(finv, denom_sh.at[dstv], add=True)
        return carry

    lax.fori_loop(0, EPW // EB, chunk, 0)
    plsc.subcore_barrier()

    @pl.when(sid < 15)
    def _wr_body():
        pltpu.sync_copy(numer_sh.at[pl.ds(sid * 624, 624)],
                        numer_hbm.at[pl.ds(cid * N + sid * 624, 624)])

    @pl.when(sid == 15)
    def _wr_tail():
        pltpu.sync_copy(numer_sh.at[pl.ds(9360, 640)],
                        numer_hbm.at[pl.ds(cid * N + 9360, 640)])

    @pl.when(sid == 0)
    def _wr():
        pltpu.sync_copy(denom_sh, denom_hbm.at[pl.ds(cid * NP1, NP1)])


def _edge_kernel(nfeat, S, src, dst, erel, emonth, crel, cmon, z2, z1):
    f = pl.kernel(
        _edge_body,
        mesh=_mesh,
        out_type=[
            jax.ShapeDtypeStruct((2 * N, D), jnp.float32),
            jax.ShapeDtypeStruct((2 * NP1,), jnp.float32),
        ],
        scratch_types=(
            [pltpu.VMEM_SHARED((N, D), jnp.float32),
             pltpu.VMEM_SHARED((NP1,), jnp.float32),
             pltpu.VMEM((EB, D), jnp.float32),
             pltpu.VMEM((EB, D), jnp.float32),
             pltpu.VMEM((EB,), jnp.int32),
             pltpu.VMEM((EB,), jnp.int32),
             pltpu.VMEM((EB,), jnp.int32),
             pltpu.VMEM((EB,), jnp.int32),
             pltpu.VMEM((EB,), jnp.float32)]
            + [pltpu.VMEM((EB,), jnp.int32)] * 10
            + [pltpu.VMEM((EB,), jnp.float32)] * 10
            + [pltpu.VMEM((3, 16), jnp.float32),
               pltpu.VMEM((3, 16), jnp.float32),
               pltpu.SemaphoreType.DMA]
        ),
    )
    return f(nfeat, S, src, dst, erel, emonth, crel, cmon, z2, z1)


def _cat_body(cemb_hbm, cs_hbm, cd_hbm, z2_hbm, z1_hbm,
              cagg_hbm, cdeg_hbm,
              cagg_sh, cdeg_sh, buf_v, csv, cdv, ones_v, sem):
    cid = lax.axis_index("c")
    sid = lax.axis_index("s")

    @pl.when(sid == 0)
    def _init():
        pltpu.sync_copy(z2_hbm, cagg_sh)
        pltpu.sync_copy(z1_hbm, cdeg_sh)

    plsc.subcore_barrier()

    for k in range(CB // 16):
        ones_v[pl.ds(16 * k, 16)] = jnp.full((16,), 1.0, jnp.float32)

    wid = sid * 2 + cid
    base0 = wid * CPW

    def chunk(i, carry):
        base = base0 + i * CB
        pltpu.sync_copy(cs_hbm.at[pl.ds(base, CB)], csv)
        pltpu.sync_copy(cd_hbm.at[pl.ds(base, CB)], cdv)
        pltpu.async_copy(cemb_hbm.at[csv], buf_v, sem).wait()
        pltpu.sync_copy(buf_v, cagg_sh.at[cdv], add=True)
        pltpu.sync_copy(ones_v, cdeg_sh.at[cdv], add=True)
        return carry

    lax.fori_loop(0, CPW // CB, chunk, 0)
    plsc.subcore_barrier()

    @pl.when(sid < 15)
    def _wrc_body():
        pltpu.sync_copy(cagg_sh.at[pl.ds(sid * 624, 624)],
                        cagg_hbm.at[pl.ds(cid * N + sid * 624, 624)])

    @pl.when(sid == 15)
    def _wrc_tail():
        pltpu.sync_copy(cagg_sh.at[pl.ds(9360, 640)],
                        cagg_hbm.at[pl.ds(cid * N + 9360, 640)])

    @pl.when(sid == 0)
    def _wr():
        pltpu.sync_copy(cdeg_sh, cdeg_hbm.at[pl.ds(cid * NP1, NP1)])


def _cat_kernel(cemb, cs, cd, z2, z1):
    f = pl.kernel(
        _cat_body,
        mesh=_mesh,
        out_type=[
            jax.ShapeDtypeStruct((2 * N, D), jnp.float32),
            jax.ShapeDtypeStruct((2 * NP1,), jnp.float32),
        ],
        scratch_types=[
            pltpu.VMEM_SHARED((N + 8, D), jnp.float32),
            pltpu.VMEM_SHARED((NP1,), jnp.float32),
            pltpu.VMEM((CB, D), jnp.float32),
            pltpu.VMEM((CB,), jnp.int32),
            pltpu.VMEM((CB,), jnp.int32),
            pltpu.VMEM((CB,), jnp.float32),
            pltpu.SemaphoreType.DMA,
        ],
    )
    return f(cemb, cs, cd, z2, z1)


def _final_body(n0_ref, n1_ref, dn_ref, c0_ref, c1_ref, cd_ref,
                w1_ref, w2_ref, b_ref, o_ref):
    dns = jnp.sum(dn_ref[...], axis=1, keepdims=True)
    rst = (n0_ref[...] + n1_ref[...]) / (dns + 1e-9)
    cds = jnp.sum(cd_ref[...], axis=1, keepdims=True)
    cat = (c0_ref[...] + c1_ref[...]) / (cds + 1e-9)
    acc = (jnp.dot(rst, w1_ref[...], preferred_element_type=jnp.float32)
           + jnp.dot(cat, w2_ref[...], preferred_element_type=jnp.float32)
           + b_ref[...])
    o_ref[...] = jnp.maximum(acc, 0.0)


def _final(n0, n1, dn, c0, c1, cd, w1, w2, b):
    nblk = 1000
    return pl.pallas_call(
        _final_body,
        grid=(N // nblk,),
        in_specs=[
            pl.BlockSpec((nblk, D), lambda i: (i, 0)),
            pl.BlockSpec((nblk, D), lambda i: (i, 0)),
            pl.BlockSpec((nblk, 2), lambda i: (i, 0)),
            pl.BlockSpec((nblk, D), lambda i: (i, 0)),
            pl.BlockSpec((nblk, D), lambda i: (i, 0)),
            pl.BlockSpec((nblk, 2), lambda i: (i, 0)),
            pl.BlockSpec((D, D), lambda i: (0, 0)),
            pl.BlockSpec((D, D), lambda i: (0, 0)),
            pl.BlockSpec((1, D), lambda i: (0, 0)),
        ],
        out_specs=pl.BlockSpec((nblk, D), lambda i: (i, 0)),
        out_shape=jax.ShapeDtypeStruct((N, D), jnp.float32),
    )(n0, n1, dn, c0, c1, cd, w1, w2, b)


def kernel(nfeat, edge_index, erel, emonth, cat_edge_index, cat_embedding,
           rel_emb, norm_emb, month_emb, W_tn, b_tn, W_tr, b_tr, W_te, b_te):
    i32 = jnp.int32
    src = edge_index[0].astype(i32)
    dst = edge_index[1].astype(i32)
    erel = erel.astype(i32)
    emonth = emonth.astype(i32)

    m = jnp.arange(12)
    mix = (month_emb[(m - 1) % 12] + month_emb[m] + month_emb[(m + 1) % 12]) / 3.0
    tn_all = mix @ W_tn.T + b_tn
    tr_all = mix @ W_tr.T + b_tr
    c1 = jnp.sum(norm_emb ** 2, 1) - 2.0
    c2 = jnp.sum(rel_emb ** 2, 1)
    c3 = jnp.sum(norm_emb * rel_emb, 1)
    d1 = jnp.sum(tn_all ** 2, 1) - 2.0
    d2 = jnp.sum(tr_all ** 2, 1)
    d3 = jnp.sum(tn_all * tr_all, 1)
    crel = jnp.zeros((3, 16), jnp.float32)
    crel = crel.at[0, :7].set(c1).at[1, :7].set(c2).at[2, :7].set(c3)
    cmon = jnp.zeros((3, 16), jnp.float32)
    cmon = cmon.at[0, :12].set(d1).at[1, :12].set(d2).at[2, :12].set(d3)

    P = jnp.zeros((D, SW), jnp.float32)
    P = P.at[:, 0:7].set(norm_emb.T).at[:, 7:14].set(rel_emb.T)
    P = P.at[:, 14:26].set(tn_all.T).at[:, 26:38].set(tr_all.T)

    S = _node_table(nfeat, P)

    z2 = jnp.zeros((N + 8, D), jnp.float32)
    z1 = jnp.zeros((NP1,), jnp.float32)

    numer, denom = _edge_kernel(nfeat, S.reshape(N * SW), src, dst,
                                erel, emonth, crel, cmon, z2, z1)

    cs = jnp.concatenate([cat_edge_index[0].astype(i32),
                          jnp.zeros((ECP - EC,), i32)])
    cd = jnp.concatenate([cat_edge_index[1].astype(i32),
                          jnp.full((ECP - EC,), N, i32)])
    cagg, cdeg = _cat_kernel(cat_embedding, cs, cd, z2, z1)

    w1 = W_te[:, :D].T
    w2 = W_te[:, D:].T
    out = _final(numer[:N], numer[N:], denom.reshape(2, NP1)[:, :N].T,
                 cagg[:N], cagg[N:], cdeg.reshape(2, NP1)[:, :N].T,
                 w1, w2, b_te.reshape(1, D))
    return out


# speedup vs baseline: 8.7205x; 1.1791x over previous
import functools
import jax
import jax.numpy as jnp
from jax import lax
from jax.experimental import pallas as pl
from jax.experimental.pallas import tpu as pltpu
from jax.experimental.pallas import tpu_sc as plsc

N = 10000
E = 320000
D = 128
NC = 500
EC = 100000
ECP = 102400
SW = 48
NW = 32
EB = 80
CB = 80
KS = 1
EPW = E // NW
CPW = ECP // NW
NP1 = 10240

_mesh = plsc.VectorSubcoreMesh(core_axis_name="c", subcore_axis_name="s")


def _stable_body(x_ref, p_ref, o_ref):
    x = x_ref[...]
    s = jnp.dot(x, p_ref[...], preferred_element_type=jnp.float32)
    nn = jnp.sum(x * x, axis=1, keepdims=True)
    col = lax.broadcasted_iota(jnp.int32, s.shape, 1)
    o_ref[...] = jnp.where(col == 38, nn, s)


def _node_table(nfeat, P):
    nblk = 1000
    return pl.pallas_call(
        _stable_body,
        grid=(N // nblk,),
        in_specs=[
            pl.BlockSpec((nblk, D), lambda i: (i, 0)),
            pl.BlockSpec((D, SW), lambda i: (0, 0)),
        ],
        out_specs=pl.BlockSpec((nblk, SW), lambda i: (i, 0)),
        out_shape=jax.ShapeDtypeStruct((N, SW), jnp.float32),
    )(nfeat, P)


def _edge_body(nfeat_hbm, sflat_hbm, src_hbm, dst_hbm, rel_hbm, mon_hbm,
               crel_hbm, cmon_hbm, z2_hbm, z1_hbm,
               numer_hbm, denom_hbm,
               numer_sh, denom_sh,
               h_v, t_v, srcv, dstv, relv, monv, finv,
               ia0, ia1, ia2, ia3, ia4, ia5, ia6, ia7, ia8, ia9,
               va0, va1, va2, va3, va4, va5, va6, va7, va8, va9,
               crel_v, cmon_v, sem):
    cid = lax.axis_index("c")
    sid = lax.axis_index("s")

    @pl.when(sid == 0)
    def _init():
        pltpu.sync_copy(z2_hbm.at[pl.ds(0, N)], numer_sh)
        pltpu.sync_copy(z1_hbm, denom_sh)

    plsc.subcore_barrier()

    pltpu.sync_copy(crel_hbm, crel_v)
    pltpu.sync_copy(cmon_hbm, cmon_v)
    c1vec = crel_v[0, :]
    c2vec = crel_v[1, :]
    c3vec = crel_v[2, :]
    d1vec = cmon_v[0, :]
    d2vec = cmon_v[1, :]
    d3vec = cmon_v[2, :]
    lane = lax.iota(jnp.int32, 16)
    p8 = lane ^ 8
    p4 = lane ^ 4
    p2 = lane ^ 2
    p1 = lane ^ 1
    zlane = lane * 0

    wid = sid * 2 + cid
    base0 = wid * EPW

    def chunk(i, carry):
        base = base0 + i * EB
        cps = []
        for k in range(KS):
            off = base + k * 80
            cps.append(pltpu.async_copy(src_hbm.at[pl.ds(off, 80)],
                                        srcv.at[k], sem))
            cps.append(pltpu.async_copy(dst_hbm.at[pl.ds(off, 80)],
                                        dstv.at[k], sem))
            cps.append(pltpu.async_copy(rel_hbm.at[pl.ds(off, 80)],
                                        relv.at[k], sem))
            cps.append(pltpu.async_copy(mon_hbm.at[pl.ds(off, 80)],
                                        monv.at[k], sem))
        for cp in cps:
            cp.wait()

        for k in range(KS):
            def bidx(g, c):
                sl = pl.ds(g * 16, 16)
                s16 = srcv[k, sl] * SW
                d16 = dstv[k, sl] * SW
                r16 = relv[k, sl]
                m16 = monv[k, sl]
                ia0[k, sl] = s16 + r16
                ia1[k, sl] = d16 + r16
                ia2[k, sl] = s16 + (r16 + 7)
                ia3[k, sl] = d16 + (r16 + 7)
                ia4[k, sl] = s16 + (m16 + 14)
                ia5[k, sl] = d16 + (m16 + 14)
                ia6[k, sl] = s16 + (m16 + 26)
                ia7[k, sl] = d16 + (m16 + 26)
                ia8[k, sl] = s16 + 38
                ia9[k, sl] = d16 + 38
                return c

            lax.fori_loop(0, 5, bidx, 0)

        cps = []
        for k in range(KS):
            ksl = pl.ds(k * 80, 80)
            cps.append(pltpu.async_copy(nfeat_hbm.at[srcv.at[k]],
                                        h_v.at[ksl], sem))
            cps.append(pltpu.async_copy(nfeat_hbm.at[dstv.at[k]],
                                        t_v.at[ksl], sem))
            for iref, vref in ((ia0, va0), (ia1, va1), (ia2, va2),
                               (ia3, va3), (ia4, va4), (ia5, va5),
                               (ia6, va6), (ia7, va7), (ia8, va8),
                               (ia9, va9)):
                cps.append(pltpu.async_copy(sflat_hbm.at[iref.at[k]],
                                            vref.at[ksl], sem))
        for cp in cps:
            cp.wait()

        def group(g, c):
            k = g // 5
            e0 = g * 16
            sl = pl.ds(e0, 16)
            slk = pl.ds((g - k * 5) * 16, 16)
            dot16 = jnp.zeros((16,), jnp.float32)
            for l in range(16):
                acc = jnp.zeros((16,), jnp.float32)
                for j in range(8):
                    dsl = pl.ds(16 * j, 16)
                    acc = acc + h_v[e0 + l, dsl] * t_v[e0 + l, dsl]
                acc = acc + acc.at[p8].get(mode="promise_in_bounds")
                acc = acc + acc.at[p4].get(mode="promise_in_bounds")
                acc = acc + acc.at[p2].get(mode="promise_in_bounds")
                acc = acc + acc.at[p1].get(mode="promise_in_bounds")
                dot16 = jnp.where(lane == l, acc, dot16)
            r16 = relv[k, slk]
            m16 = monv[k, slk]
            c1v = c1vec.at[r16].get(mode="promise_in_bounds")
            c2v = c2vec.at[r16].get(mode="promise_in_bounds")
            c3v = c3vec.at[r16].get(mode="promise_in_bounds")
            d1v = d1vec.at[m16].get(mode="promise_in_bounds")
            d2v = d2vec.at[m16].get(mode="promise_in_bounds")
            d3v = d3vec.at[m16].get(mode="promise_in_bounds")
            q = va8[sl] + va9[sl] - 2.0 * dot16
            su = va0[sl] - va1[sl]
            url = va2[sl] - va3[sl]
            stn = va4[sl] - va5[sl]
            utr = va6[sl] - va7[sl]
            dist1 = q + su * su * c1v + c2v + 2.0 * url - 2.0 * su * c3v
            dist2 = q + stn * stn * d1v + d2v + 2.0 * utr - 2.0 * stn * d3v
            fin16 = jnp.exp(-(dist1 + dist2))
            finv[k, slk] = fin16
            for l in range(16):
                fb = fin16.at[zlane + l].get(mode="promise_in_bounds")
                for j in range(8):
                    dsl = pl.ds(16 * j, 16)
                    h_v[e0 + l, dsl] = h_v[e0 + l, dsl] * fb
            return c

        lax.fori_loop(0, EB // 16, group, 0)
        cps = []
        for k in range(KS):
            ksl = pl.ds(k * 80, 80)
            cps.append(pltpu.async_copy(h_v.at[ksl],
                                        numer_sh.at[dstv.at[k]], sem,
                                        add=True))
            cps.append(pltpu.async_copy(finv.at[k],
                                        denom_sh.at[dstv.at[k]], sem,
                                        add=True))
        for cp in cps:
            cp.wait()
        return carry

    lax.fori_loop(0, EPW // EB, chunk, 0)
    plsc.subcore_barrier()

    @pl.when(sid < 15)
    def _wr_body():
        pltpu.sync_copy(numer_sh.at[pl.ds(sid * 624, 624)],
                        numer_hbm.at[pl.ds(cid * N + sid * 624, 624)])

    @pl.when(sid == 15)
    def _wr_tail():
        pltpu.sync_copy(numer_sh.at[pl.ds(9360, 640)],
                        numer_hbm.at[pl.ds(cid * N + 9360, 640)])

    @pl.when(sid == 0)
    def _wr():
        pltpu.sync_copy(denom_sh, denom_hbm.at[pl.ds(cid * NP1, NP1)])


def _edge_kernel(nfeat, S, src, dst, erel, emonth, crel, cmon, z2, z1):
    f = pl.kernel(
        _edge_body,
        mesh=_mesh,
        out_type=[
            jax.ShapeDtypeStruct((2 * N, D), jnp.float32),
            jax.ShapeDtypeStruct((2 * NP1,), jnp.float32),
        ],
        scratch_types=(
            [pltpu.VMEM_SHARED((N, D), jnp.float32),
             pltpu.VMEM_SHARED((NP1,), jnp.float32),
             pltpu.VMEM((EB, D), jnp.float32),
             pltpu.VMEM((EB, D), jnp.float32),
             pltpu.VMEM((KS, 80), jnp.int32),
             pltpu.VMEM((KS, 80), jnp.int32),
             pltpu.VMEM((KS, 80), jnp.int32),
             pltpu.VMEM((KS, 80), jnp.int32),
             pltpu.VMEM((KS, 80), jnp.float32)]
            + [pltpu.VMEM((KS, 80), jnp.int32)] * 10
            + [pltpu.VMEM((EB,), jnp.float32)] * 10
            + [pltpu.VMEM((3, 16), jnp.float32),
               pltpu.VMEM((3, 16), jnp.float32),
               pltpu.SemaphoreType.DMA]
        ),
    )
    return f(nfeat, S, src, dst, erel, emonth, crel, cmon, z2, z1)


def _cat_body(cemb_hbm, cs_hbm, cd_hbm, z2_hbm, z1_hbm,
              cagg_hbm, cdeg_hbm,
              cagg_sh, cdeg_sh, buf_v, csv, cdv, ones_v, sem):
    cid = lax.axis_index("c")
    sid = lax.axis_index("s")

    @pl.when(sid == 0)
    def _init():
        pltpu.sync_copy(z2_hbm, cagg_sh)
        pltpu.sync_copy(z1_hbm, cdeg_sh)

    plsc.subcore_barrier()

    def fill_ones(g, c):
        ones_v[pl.ds(g * 16, 16)] = jnp.full((16,), 1.0, jnp.float32)
        return c

    lax.fori_loop(0, CB // 16, fill_ones, 0)

    wid = sid * 2 + cid
    base0 = wid * CPW

    def chunk(i, carry):
        base = base0 + i * CB
        cps = []
        for k in range(KS):
            off = base + k * 80
            cps.append(pltpu.async_copy(cs_hbm.at[pl.ds(off, 80)],
                                        csv.at[k], sem))
            cps.append(pltpu.async_copy(cd_hbm.at[pl.ds(off, 80)],
                                        cdv.at[k], sem))
        for cp in cps:
            cp.wait()
        cps = []
        for k in range(KS):
            cps.append(pltpu.async_copy(cemb_hbm.at[csv.at[k]],
                                        buf_v.at[pl.ds(k * 80, 80)], sem))
        for cp in cps:
            cp.wait()
        cps = []
        for k in range(KS):
            cps.append(pltpu.async_copy(buf_v.at[pl.ds(k * 80, 80)],
                                        cagg_sh.at[cdv.at[k]], sem,
                                        add=True))
            cps.append(pltpu.async_copy(ones_v.at[pl.ds(k * 80, 80)],
                                        cdeg_sh.at[cdv.at[k]], sem,
                                        add=True))
        for cp in cps:
            cp.wait()
        return carry

    lax.fori_loop(0, CPW // CB, chunk, 0)
    plsc.subcore_barrier()

    @pl.when(sid < 15)
    def _wrc_body():
        pltpu.sync_copy(cagg_sh.at[pl.ds(sid * 624, 624)],
                        cagg_hbm.at[pl.ds(cid * N + sid * 624, 624)])

    @pl.when(sid == 15)
    def _wrc_tail():
        pltpu.sync_copy(cagg_sh.at[pl.ds(9360, 640)],
                        cagg_hbm.at[pl.ds(cid * N + 9360, 640)])

    @pl.when(sid == 0)
    def _wr():
        pltpu.sync_copy(cdeg_sh, cdeg_hbm.at[pl.ds(cid * NP1, NP1)])


def _cat_kernel(cemb, cs, cd, z2, z1):
    f = pl.kernel(
        _cat_body,
        mesh=_mesh,
        out_type=[
            jax.ShapeDtypeStruct((2 * N, D), jnp.float32),
            jax.ShapeDtypeStruct((2 * NP1,), jnp.float32),
        ],
        scratch_types=[
            pltpu.VMEM_SHARED((N + 8, D), jnp.float32),
            pltpu.VMEM_SHARED((NP1,), jnp.float32),
            pltpu.VMEM((CB, D), jnp.float32),
            pltpu.VMEM((KS, 80), jnp.int32),
            pltpu.VMEM((KS, 80), jnp.int32),
            pltpu.VMEM((CB,), jnp.float32),
            pltpu.SemaphoreType.DMA,
        ],
    )
    return f(cemb, cs, cd, z2, z1)


def _final_body(n0_ref, n1_ref, dn_ref, c0_ref, c1_ref, cd_ref,
                w1_ref, w2_ref, b_ref, o_ref):
    dns = jnp.sum(dn_ref[...], axis=1, keepdims=True)
    rst = (n0_ref[...] + n1_ref[...]) / (dns + 1e-9)
    cds = jnp.sum(cd_ref[...], axis=1, keepdims=True)
    cat = (c0_ref[...] + c1_ref[...]) / (cds + 1e-9)
    acc = (jnp.dot(rst, w1_ref[...], preferred_element_type=jnp.float32)
           + jnp.dot(cat, w2_ref[...], preferred_element_type=jnp.float32)
           + b_ref[...])
    o_ref[...] = jnp.maximum(acc, 0.0)


def _final(n0, n1, dn, c0, c1, cd, w1, w2, b):
    nblk = 1000
    return pl.pallas_call(
        _final_body,
        grid=(N // nblk,),
        in_specs=[
            pl.BlockSpec((nblk, D), lambda i: (i, 0)),
            pl.BlockSpec((nblk, D), lambda i: (i, 0)),
            pl.BlockSpec((nblk, 2), lambda i: (i, 0)),
            pl.BlockSpec((nblk, D), lambda i: (i, 0)),
            pl.BlockSpec((nblk, D), lambda i: (i, 0)),
            pl.BlockSpec((nblk, 2), lambda i: (i, 0)),
            pl.BlockSpec((D, D), lambda i: (0, 0)),
            pl.BlockSpec((D, D), lambda i: (0, 0)),
            pl.BlockSpec((1, D), lambda i: (0, 0)),
        ],
        out_specs=pl.BlockSpec((nblk, D), lambda i: (i, 0)),
        out_shape=jax.ShapeDtypeStruct((N, D), jnp.float32),
    )(n0, n1, dn, c0, c1, cd, w1, w2, b)


def kernel(nfeat, edge_index, erel, emonth, cat_edge_index, cat_embedding,
           rel_emb, norm_emb, month_emb, W_tn, b_tn, W_tr, b_tr, W_te, b_te):
    i32 = jnp.int32
    src = edge_index[0].astype(i32)
    dst = edge_index[1].astype(i32)
    erel = erel.astype(i32)
    emonth = emonth.astype(i32)

    m = jnp.arange(12)
    mix = (month_emb[(m - 1) % 12] + month_emb[m] + month_emb[(m + 1) % 12]) / 3.0
    tn_all = mix @ W_tn.T + b_tn
    tr_all = mix @ W_tr.T + b_tr
    c1 = jnp.sum(norm_emb ** 2, 1) - 2.0
    c2 = jnp.sum(rel_emb ** 2, 1)
    c3 = jnp.sum(norm_emb * rel_emb, 1)
    d1 = jnp.sum(tn_all ** 2, 1) - 2.0
    d2 = jnp.sum(tr_all ** 2, 1)
    d3 = jnp.sum(tn_all * tr_all, 1)
    crel = jnp.zeros((3, 16), jnp.float32)
    crel = crel.at[0, :7].set(c1).at[1, :7].set(c2).at[2, :7].set(c3)
    cmon = jnp.zeros((3, 16), jnp.float32)
    cmon = cmon.at[0, :12].set(d1).at[1, :12].set(d2).at[2, :12].set(d3)

    P = jnp.zeros((D, SW), jnp.float32)
    P = P.at[:, 0:7].set(norm_emb.T).at[:, 7:14].set(rel_emb.T)
    P = P.at[:, 14:26].set(tn_all.T).at[:, 26:38].set(tr_all.T)

    S = _node_table(nfeat, P)

    z2 = jnp.zeros((N + 8, D), jnp.float32)
    z1 = jnp.zeros((NP1,), jnp.float32)

    numer, denom = _edge_kernel(nfeat, S.reshape(N * SW), src, dst,
                                erel, emonth, crel, cmon, z2, z1)

    cs = jnp.concatenate([cat_edge_index[0].astype(i32),
                          jnp.zeros((ECP - EC,), i32)])
    cd = jnp.concatenate([cat_edge_index[1].astype(i32),
                          jnp.full((ECP - EC,), N, i32)])
    cagg, cdeg = _cat_kernel(cat_embedding, cs, cd, z2, z1)

    w1 = W_te[:, :D].T
    w2 = W_te[:, D:].T
    out = _final(numer[:N], numer[N:], denom.reshape(2, NP1)[:, :N].T,
                 cagg[:N], cagg[N:], cdeg.reshape(2, NP1)[:, :N].T,
                 w1, w2, b_te.reshape(1, D))
    return out


# speedup vs baseline: 8.8603x; 1.0160x over previous
import functools
import jax
import jax.numpy as jnp
from jax import lax
from jax.experimental import pallas as pl
from jax.experimental.pallas import tpu as pltpu
from jax.experimental.pallas import tpu_sc as plsc

N = 10000
E = 320000
D = 128
NC = 500
EC = 100000
ECP = 102400
SW = 48
NW = 32
EB = 80
CB = 160
KS = 1
CKS = 2
EPW = E // NW
CPW = ECP // NW
NP1 = 10240

_mesh = plsc.VectorSubcoreMesh(core_axis_name="c", subcore_axis_name="s")


def _stable_body(x_ref, p_ref, o_ref):
    x = x_ref[...]
    s = jnp.dot(x, p_ref[...], preferred_element_type=jnp.float32)
    nn = jnp.sum(x * x, axis=1, keepdims=True)
    col = lax.broadcasted_iota(jnp.int32, s.shape, 1)
    o_ref[...] = jnp.where(col == 38, nn, s)


def _node_table(nfeat, P):
    nblk = 1000
    return pl.pallas_call(
        _stable_body,
        grid=(N // nblk,),
        in_specs=[
            pl.BlockSpec((nblk, D), lambda i: (i, 0)),
            pl.BlockSpec((D, SW), lambda i: (0, 0)),
        ],
        out_specs=pl.BlockSpec((nblk, SW), lambda i: (i, 0)),
        out_shape=jax.ShapeDtypeStruct((N, SW), jnp.float32),
    )(nfeat, P)


def _edge_body(nfeat_hbm, sflat_hbm, src_hbm, dst_hbm, rel_hbm, mon_hbm,
               crel_hbm, cmon_hbm, z2_hbm, z1_hbm,
               numer_hbm, denom_hbm,
               numer_sh, denom_sh,
               h_v, t_v, srcv, dstv, relv, monv, finv,
               ia0, ia1, ia2, ia3, ia4, ia5, ia6, ia7, ia8, ia9,
               va0, va1, va2, va3, va4, va5, va6, va7, va8, va9,
               crel_v, cmon_v, sem):
    cid = lax.axis_index("c")
    sid = lax.axis_index("s")

    @pl.when(sid == 0)
    def _init():
        pltpu.sync_copy(z2_hbm.at[pl.ds(0, N)], numer_sh)
        pltpu.sync_copy(z1_hbm, denom_sh)

    plsc.subcore_barrier()

    pltpu.sync_copy(crel_hbm, crel_v)
    pltpu.sync_copy(cmon_hbm, cmon_v)
    c1vec = crel_v[0, :]
    c2vec = crel_v[1, :]
    c3vec = crel_v[2, :]
    d1vec = cmon_v[0, :]
    d2vec = cmon_v[1, :]
    d3vec = cmon_v[2, :]
    lane = lax.iota(jnp.int32, 16)
    p8 = lane ^ 8
    p4 = lane ^ 4
    p2 = lane ^ 2
    p1 = lane ^ 1
    zlane = lane * 0

    wid = sid * 2 + cid
    base0 = wid * EPW

    def chunk(i, carry):
        base = base0 + i * EB
        cps = []
        for k in range(KS):
            off = base + k * 80
            cps.append(pltpu.async_copy(src_hbm.at[pl.ds(off, 80)],
                                        srcv.at[k], sem))
            cps.append(pltpu.async_copy(dst_hbm.at[pl.ds(off, 80)],
                                        dstv.at[k], sem))
            cps.append(pltpu.async_copy(rel_hbm.at[pl.ds(off, 80)],
                                        relv.at[k], sem))
            cps.append(pltpu.async_copy(mon_hbm.at[pl.ds(off, 80)],
                                        monv.at[k], sem))
        for cp in cps:
            cp.wait()

        for k in range(KS):
            def bidx(g, c):
                sl = pl.ds(g * 16, 16)
                s16 = srcv[k, sl] * SW
                d16 = dstv[k, sl] * SW
                r16 = relv[k, sl]
                m16 = monv[k, sl]
                ia0[k, sl] = s16 + r16
                ia1[k, sl] = d16 + r16
                ia2[k, sl] = s16 + (r16 + 7)
                ia3[k, sl] = d16 + (r16 + 7)
                ia4[k, sl] = s16 + (m16 + 14)
                ia5[k, sl] = d16 + (m16 + 14)
                ia6[k, sl] = s16 + (m16 + 26)
                ia7[k, sl] = d16 + (m16 + 26)
                ia8[k, sl] = s16 + 38
                ia9[k, sl] = d16 + 38
                return c

            lax.fori_loop(0, 5, bidx, 0)

        cps = []
        for k in range(KS):
            ksl = pl.ds(k * 80, 80)
            cps.append(pltpu.async_copy(nfeat_hbm.at[srcv.at[k]],
                                        h_v.at[ksl], sem))
            cps.append(pltpu.async_copy(nfeat_hbm.at[dstv.at[k]],
                                        t_v.at[ksl], sem))
            for iref, vref in ((ia0, va0), (ia1, va1), (ia2, va2),
                               (ia3, va3), (ia4, va4), (ia5, va5),
                               (ia6, va6), (ia7, va7), (ia8, va8),
                               (ia9, va9)):
                cps.append(pltpu.async_copy(sflat_hbm.at[iref.at[k]],
                                            vref.at[ksl], sem))
        for cp in cps:
            cp.wait()

        def group(g, c):
            k = g // 5
            e0 = g * 16
            sl = pl.ds(e0, 16)
            slk = pl.ds((g - k * 5) * 16, 16)
            dot16 = jnp.zeros((16,), jnp.float32)
            for l in range(16):
                acc = jnp.zeros((16,), jnp.float32)
                for j in range(8):
                    dsl = pl.ds(16 * j, 16)
                    acc = acc + h_v[e0 + l, dsl] * t_v[e0 + l, dsl]
                acc = acc + acc.at[p8].get(mode="promise_in_bounds")
                acc = acc + acc.at[p4].get(mode="promise_in_bounds")
                acc = acc + acc.at[p2].get(mode="promise_in_bounds")
                acc = acc + acc.at[p1].get(mode="promise_in_bounds")
                dot16 = jnp.where(lane == l, acc, dot16)
            r16 = relv[k, slk]
            m16 = monv[k, slk]
            c1v = c1vec.at[r16].get(mode="promise_in_bounds")
            c2v = c2vec.at[r16].get(mode="promise_in_bounds")
            c3v = c3vec.at[r16].get(mode="promise_in_bounds")
            d1v = d1vec.at[m16].get(mode="promise_in_bounds")
            d2v = d2vec.at[m16].get(mode="promise_in_bounds")
            d3v = d3vec.at[m16].get(mode="promise_in_bounds")
            q = va8[sl] + va9[sl] - 2.0 * dot16
            su = va0[sl] - va1[sl]
            url = va2[sl] - va3[sl]
            stn = va4[sl] - va5[sl]
            utr = va6[sl] - va7[sl]
            dist1 = q + su * su * c1v + c2v + 2.0 * url - 2.0 * su * c3v
            dist2 = q + stn * stn * d1v + d2v + 2.0 * utr - 2.0 * stn * d3v
            fin16 = jnp.exp(-(dist1 + dist2))
            finv[k, slk] = fin16
            for l in range(16):
                fb = fin16.at[zlane + l].get(mode="promise_in_bounds")
                for j in range(8):
                    dsl = pl.ds(16 * j, 16)
                    h_v[e0 + l, dsl] = h_v[e0 + l, dsl] * fb
            return c

        lax.fori_loop(0, EB // 16, group, 0)
        cps = []
        for k in range(KS):
            ksl = pl.ds(k * 80, 80)
            cps.append(pltpu.async_copy(h_v.at[ksl],
                                        numer_sh.at[dstv.at[k]], sem,
                                        add=True))
            cps.append(pltpu.async_copy(finv.at[k],
                                        denom_sh.at[dstv.at[k]], sem,
                                        add=True))
        for cp in cps:
            cp.wait()
        return carry

    lax.fori_loop(0, EPW // EB, chunk, 0)
    plsc.subcore_barrier()

    @pl.when(sid < 15)
    def _wr_body():
        pltpu.sync_copy(numer_sh.at[pl.ds(sid * 624, 624)],
                        numer_hbm.at[pl.ds(cid * N + sid * 624, 624)])

    @pl.when(sid == 15)
    def _wr_tail():
        pltpu.sync_copy(numer_sh.at[pl.ds(9360, 640)],
                        numer_hbm.at[pl.ds(cid * N + 9360, 640)])

    @pl.when(sid == 0)
    def _wr():
        pltpu.sync_copy(denom_sh, denom_hbm.at[pl.ds(cid * NP1, NP1)])


def _edge_kernel(nfeat, S, src, dst, erel, emonth, crel, cmon, z2, z1):
    f = pl.kernel(
        _edge_body,
        mesh=_mesh,
        out_type=[
            jax.ShapeDtypeStruct((2 * N, D), jnp.float32),
            jax.ShapeDtypeStruct((2 * NP1,), jnp.float32),
        ],
        scratch_types=(
            [pltpu.VMEM_SHARED((N, D), jnp.float32),
             pltpu.VMEM_SHARED((NP1,), jnp.float32),
             pltpu.VMEM((EB, D), jnp.float32),
             pltpu.VMEM((EB, D), jnp.float32),
             pltpu.VMEM((KS, 80), jnp.int32),
             pltpu.VMEM((KS, 80), jnp.int32),
             pltpu.VMEM((KS, 80), jnp.int32),
             pltpu.VMEM((KS, 80), jnp.int32),
             pltpu.VMEM((KS, 80), jnp.float32)]
            + [pltpu.VMEM((KS, 80), jnp.int32)] * 10
            + [pltpu.VMEM((EB,), jnp.float32)] * 10
            + [pltpu.VMEM((3, 16), jnp.float32),
               pltpu.VMEM((3, 16), jnp.float32),
               pltpu.SemaphoreType.DMA]
        ),
    )
    return f(nfeat, S, src, dst, erel, emonth, crel, cmon, z2, z1)


def _cat_body(cemb_hbm, cs_hbm, cd_hbm, z2_hbm, z1_hbm,
              cagg_hbm, cdeg_hbm,
              cagg_sh, cdeg_sh, buf_v, csv, cdv, ones_v, sem):
    cid = lax.axis_index("c")
    sid = lax.axis_index("s")

    @pl.when(sid == 0)
    def _init():
        pltpu.sync_copy(z2_hbm, cagg_sh)
        pltpu.sync_copy(z1_hbm, cdeg_sh)

    plsc.subcore_barrier()

    def fill_ones(g, c):
        ones_v[pl.ds(g * 16, 16)] = jnp.full((16,), 1.0, jnp.float32)
        return c

    lax.fori_loop(0, CB // 16, fill_ones, 0)

    wid = sid * 2 + cid
    base0 = wid * CPW

    def chunk(i, carry):
        base = base0 + i * CB
        cps = []
        for k in range(CKS):
            off = base + k * 80
            cps.append(pltpu.async_copy(cs_hbm.at[pl.ds(off, 80)],
                                        csv.at[k], sem))
            cps.append(pltpu.async_copy(cd_hbm.at[pl.ds(off, 80)],
                                        cdv.at[k], sem))
        for cp in cps:
            cp.wait()
        cps = []
        for k in range(CKS):
            cps.append(pltpu.async_copy(cemb_hbm.at[csv.at[k]],
                                        buf_v.at[pl.ds(k * 80, 80)], sem))
        for cp in cps:
            cp.wait()
        cps = []
        for k in range(CKS):
            cps.append(pltpu.async_copy(buf_v.at[pl.ds(k * 80, 80)],
                                        cagg_sh.at[cdv.at[k]], sem,
                                        add=True))
            cps.append(pltpu.async_copy(ones_v.at[pl.ds(k * 80, 80)],
                                        cdeg_sh.at[cdv.at[k]], sem,
                                        add=True))
        for cp in cps:
            cp.wait()
        return carry

    lax.fori_loop(0, CPW // CB, chunk, 0)
    plsc.subcore_barrier()

    @pl.when(sid < 15)
    def _wrc_body():
        pltpu.sync_copy(cagg_sh.at[pl.ds(sid * 624, 624)],
                        cagg_hbm.at[pl.ds(cid * N + sid * 624, 624)])

    @pl.when(sid == 15)
    def _wrc_tail():
        pltpu.sync_copy(cagg_sh.at[pl.ds(9360, 640)],
                        cagg_hbm.at[pl.ds(cid * N + 9360, 640)])

    @pl.when(sid == 0)
    def _wr():
        pltpu.sync_copy(cdeg_sh, cdeg_hbm.at[pl.ds(cid * NP1, NP1)])


def _cat_kernel(cemb, cs, cd, z2, z1):
    f = pl.kernel(
        _cat_body,
        mesh=_mesh,
        out_type=[
            jax.ShapeDtypeStruct((2 * N, D), jnp.float32),
            jax.ShapeDtypeStruct((2 * NP1,), jnp.float32),
        ],
        scratch_types=[
            pltpu.VMEM_SHARED((N + 8, D), jnp.float32),
            pltpu.VMEM_SHARED((NP1,), jnp.float32),
            pltpu.VMEM((CB, D), jnp.float32),
            pltpu.VMEM((CKS, 80), jnp.int32),
            pltpu.VMEM((CKS, 80), jnp.int32),
            pltpu.VMEM((CB,), jnp.float32),
            pltpu.SemaphoreType.DMA,
        ],
    )
    return f(cemb, cs, cd, z2, z1)


def _final_body(n0_ref, n1_ref, dn_ref, c0_ref, c1_ref, cd_ref,
                w1_ref, w2_ref, b_ref, o_ref):
    dns = jnp.sum(dn_ref[...], axis=1, keepdims=True)
    rst = (n0_ref[...] + n1_ref[...]) / (dns + 1e-9)
    cds = jnp.sum(cd_ref[...], axis=1, keepdims=True)
    cat = (c0_ref[...] + c1_ref[...]) / (cds + 1e-9)
    acc = (jnp.dot(rst, w1_ref[...], preferred_element_type=jnp.float32)
           + jnp.dot(cat, w2_ref[...], preferred_element_type=jnp.float32)
           + b_ref[...])
    o_ref[...] = jnp.maximum(acc, 0.0)


def _final(n0, n1, dn, c0, c1, cd, w1, w2, b):
    nblk = 1000
    return pl.pallas_call(
        _final_body,
        grid=(N // nblk,),
        in_specs=[
            pl.BlockSpec((nblk, D), lambda i: (i, 0)),
            pl.BlockSpec((nblk, D), lambda i: (i, 0)),
            pl.BlockSpec((nblk, 2), lambda i: (i, 0)),
            pl.BlockSpec((nblk, D), lambda i: (i, 0)),
            pl.BlockSpec((nblk, D), lambda i: (i, 0)),
            pl.BlockSpec((nblk, 2), lambda i: (i, 0)),
            pl.BlockSpec((D, D), lambda i: (0, 0)),
            pl.BlockSpec((D, D), lambda i: (0, 0)),
            pl.BlockSpec((1, D), lambda i: (0, 0)),
        ],
        out_specs=pl.BlockSpec((nblk, D), lambda i: (i, 0)),
        out_shape=jax.ShapeDtypeStruct((N, D), jnp.float32),
    )(n0, n1, dn, c0, c1, cd, w1, w2, b)


def kernel(nfeat, edge_index, erel, emonth, cat_edge_index, cat_embedding,
           rel_emb, norm_emb, month_emb, W_tn, b_tn, W_tr, b_tr, W_te, b_te):
    i32 = jnp.int32
    src = edge_index[0].astype(i32)
    dst = edge_index[1].astype(i32)
    erel = erel.astype(i32)
    emonth = emonth.astype(i32)

    m = jnp.arange(12)
    mix = (month_emb[(m - 1) % 12] + month_emb[m] + month_emb[(m + 1) % 12]) / 3.0
    tn_all = mix @ W_tn.T + b_tn
    tr_all = mix @ W_tr.T + b_tr
    c1 = jnp.sum(norm_emb ** 2, 1) - 2.0
    c2 = jnp.sum(rel_emb ** 2, 1)
    c3 = jnp.sum(norm_emb * rel_emb, 1)
    d1 = jnp.sum(tn_all ** 2, 1) - 2.0
    d2 = jnp.sum(tr_all ** 2, 1)
    d3 = jnp.sum(tn_all * tr_all, 1)
    crel = jnp.zeros((3, 16), jnp.float32)
    crel = crel.at[0, :7].set(c1).at[1, :7].set(c2).at[2, :7].set(c3)
    cmon = jnp.zeros((3, 16), jnp.float32)
    cmon = cmon.at[0, :12].set(d1).at[1, :12].set(d2).at[2, :12].set(d3)

    P = jnp.zeros((D, SW), jnp.float32)
    P = P.at[:, 0:7].set(norm_emb.T).at[:, 7:14].set(rel_emb.T)
    P = P.at[:, 14:26].set(tn_all.T).at[:, 26:38].set(tr_all.T)

    S = _node_table(nfeat, P)

    z2 = jnp.zeros((N + 8, D), jnp.float32)
    z1 = jnp.zeros((NP1,), jnp.float32)

    numer, denom = _edge_kernel(nfeat, S.reshape(N * SW), src, dst,
                                erel, emonth, crel, cmon, z2, z1)

    cs = jnp.concatenate([cat_edge_index[0].astype(i32),
                          jnp.zeros((ECP - EC,), i32)])
    cd = jnp.concatenate([cat_edge_index[1].astype(i32),
                          jnp.full((ECP - EC,), N, i32)])
    cagg, cdeg = _cat_kernel(cat_embedding, cs, cd, z2, z1)

    w1 = W_te[:, :D].T
    w2 = W_te[:, D:].T
    out = _final(numer[:N], numer[N:], denom.reshape(2, NP1)[:, :N].T,
                 cagg[:N], cagg[N:], cdeg.reshape(2, NP1)[:, :N].T,
                 w1, w2, b_te.reshape(1, D))
    return out
